# Initial kernel scaffold; baseline (speedup 1.0000x reference)
import jax
import jax.numpy as jnp
from jax.experimental import pallas as pl


def kernel(x_bc, vocab_table, pos_table):
    raise NotImplementedError("write your pallas kernel here")


# speedup vs baseline: 2.4772x; 2.4772x over previous
import functools

import jax
import jax.numpy as jnp
from jax import lax
from jax.experimental import pallas as pl
from jax.experimental.pallas import tpu as pltpu
from jax.experimental.pallas import tpu_sc as plsc

VOCAB = 100000
CTX = 200
DIM = 64
BATCH = 4096
SEQ = 200

N = BATCH * SEQ
NC = 2
NS = 16
NW = NC * NS
R = N // NW
IDXW = 128
IDX_ROWS = R // IDXW
CHUNK = 512
IDX_PER_CHUNK = CHUNK // IDXW
NCHUNK = R // CHUNK
LANES = 16
DSEG = DIM // LANES

_mesh = plsc.VectorSubcoreMesh(core_axis_name="c", subcore_axis_name="s")


@functools.partial(
    pl.kernel,
    mesh=_mesh,
    compiler_params=pltpu.CompilerParams(use_tc_tiling_on_sc=False),
    out_type=jax.ShapeDtypeStruct((N, DIM), jnp.float32),
    scratch_types=[
        pltpu.VMEM((IDX_ROWS, IDXW), jnp.int32),
        pltpu.VMEM((CTX, DIM), jnp.float32),
        pltpu.VMEM((CHUNK, DIM), jnp.float32),
        pltpu.SemaphoreType.DMA,
    ],
)
def _embed(x_hbm, vocab_hbm, pos_hbm, out_hbm, idx_v, pos_v, rows_v, sem):
    cid = lax.axis_index("c")
    sid = lax.axis_index("s")
    wid = sid * NC + cid
    base = wid * R

    pltpu.sync_copy(x_hbm.at[pl.ds(wid * IDX_ROWS, IDX_ROWS)], idx_v)
    pltpu.sync_copy(pos_hbm, pos_v)

    def chunk_body(k, c0):
        copies = []
        for j in range(IDX_PER_CHUNK):
            copies.append(
                pltpu.async_copy(
                    vocab_hbm.at[idx_v.at[k * IDX_PER_CHUNK + j]],
                    rows_v.at[pl.ds(j * IDXW, IDXW)],
                    sem,
                )
            )
        for cp in copies:
            cp.wait()

        def add_body(i, c):
            for j in range(DSEG):
                sl = pl.ds(j * LANES, LANES)
                rows_v[i, sl] = rows_v[i, sl] + pos_v[c, sl]
            c = c + 1
            return lax.select(c >= CTX, c - CTX, c)

        lax.fori_loop(0, CHUNK, add_body, c0, unroll=False)

        pltpu.sync_copy(rows_v, out_hbm.at[pl.ds(base + k * CHUNK, CHUNK)])
        c0 = c0 + CHUNK % CTX
        return lax.select(c0 >= CTX, c0 - CTX, c0)

    lax.fori_loop(0, NCHUNK, chunk_body, jnp.int32(0), unroll=False)


def kernel(x_bc, vocab_table, pos_table):
    x_flat = x_bc.astype(jnp.int32).reshape(N // IDXW, IDXW)
    out = _embed(x_flat, vocab_table, pos_table)
    return out.reshape(BATCH, SEQ, DIM)


# speedup vs baseline: 3.3748x; 1.3623x over previous
import functools

import jax
import jax.numpy as jnp
from jax import lax
from jax.experimental import pallas as pl
from jax.experimental.pallas import tpu as pltpu
from jax.experimental.pallas import tpu_sc as plsc

VOCAB = 100000
CTX = 200
DIM = 64
BATCH = 4096
SEQ = 200

N = BATCH * SEQ
NC = 2
NS = 16
NW = NC * NS
R = N // NW
IDXW = 80
IDX_ROWS = R // IDXW
CHUNK = 2 * CTX
IDX_PER_CHUNK = CHUNK // IDXW
NCHUNK = R // CHUNK
HALF_PAIRS = NCHUNK // 2
LANES = 16
DSEG = DIM // LANES

_mesh = plsc.VectorSubcoreMesh(core_axis_name="c", subcore_axis_name="s")


@functools.partial(
    pl.kernel,
    mesh=_mesh,
    compiler_params=pltpu.CompilerParams(use_tc_tiling_on_sc=False),
    out_type=jax.ShapeDtypeStruct((N, DIM), jnp.float32),
    scratch_types=[
        pltpu.VMEM((IDX_ROWS, IDXW), jnp.int32),
        pltpu.VMEM((CTX, DIM), jnp.float32),
        pltpu.VMEM((CHUNK, DIM), jnp.float32),
        pltpu.VMEM((CHUNK, DIM), jnp.float32),
        pltpu.SemaphoreType.DMA,
        pltpu.SemaphoreType.DMA,
        pltpu.SemaphoreType.DMA,
        pltpu.SemaphoreType.DMA,
    ],
)
def _embed(x_hbm, vocab_hbm, pos_hbm, out_hbm,
           idx_v, pos_v, rows0, rows1, sg0, sg1, ss0, ss1):
    cid = lax.axis_index("c")
    sid = lax.axis_index("s")
    wid = sid * NC + cid
    base = wid * R

    def fire_gathers(k, rows, sem):
        for j in range(IDX_PER_CHUNK):
            pltpu.async_copy(
                vocab_hbm.at[idx_v.at[k * IDX_PER_CHUNK + j]],
                rows.at[pl.ds(j * IDXW, IDXW)],
                sem,
            )

    def wait_gathers(rows, sem):
        pltpu.make_async_copy(vocab_hbm.at[pl.ds(0, CHUNK)], rows, sem).wait()

    def fire_scatter(k, rows, sem):
        pltpu.async_copy(rows, out_hbm.at[pl.ds(base + k * CHUNK, CHUNK)], sem)

    def wait_scatter(rows, sem):
        pltpu.make_async_copy(rows, out_hbm.at[pl.ds(0, CHUNK)], sem).wait()

    def add_pos(rows):
        def body(i, carry):
            for j in range(DSEG):
                sl = pl.ds(j * LANES, LANES)
                p = pos_v[i, sl]
                rows[i, sl] = rows[i, sl] + p
                rows[i + CTX, sl] = rows[i + CTX, sl] + p
            return carry
        lax.fori_loop(0, CTX, body, jnp.int32(0), unroll=2)

    pltpu.sync_copy(x_hbm.at[pl.ds(wid * IDX_ROWS, IDX_ROWS)], idx_v)
    pltpu.sync_copy(pos_hbm, pos_v)

    fire_gathers(0, rows0, sg0)

    def round_body(g, carry):
        k = 2 * g
        wait_gathers(rows0, sg0)

        @pl.when(g > 0)
        def _():
            wait_scatter(rows1, ss1)
        fire_gathers(k + 1, rows1, sg1)

        add_pos(rows0)
        fire_scatter(k, rows0, ss0)

        wait_gathers(rows1, sg1)

        @pl.when(g < HALF_PAIRS - 1)
        def _():
            wait_scatter(rows0, ss0)
            fire_gathers(k + 2, rows0, sg0)

        add_pos(rows1)
        fire_scatter(k + 1, rows1, ss1)
        return carry

    lax.fori_loop(0, HALF_PAIRS, round_body, jnp.int32(0), unroll=False)

    wait_scatter(rows0, ss0)
    wait_scatter(rows1, ss1)


def kernel(x_bc, vocab_table, pos_table):
    x_flat = x_bc.astype(jnp.int32).reshape(N // IDXW, IDXW)
    out = _embed(x_flat, vocab_table, pos_table)
    return out.reshape(BATCH, SEQ, DIM)
